# baseline (speedup 1.0000x reference)
import functools

import jax
import jax.numpy as jnp
from jax import lax
from jax.experimental import pallas as pl
from jax.experimental.pallas import tpu as pltpu
from jax.experimental.pallas import tpu_sc as plsc

T, N, F_IN = 16, 10000, 256
H1, H2, LH, A = 64, 512, 512, 4
E = 160000

P = T // 2
W = 2 * H1
NC, NS = 2, 16
NW = NC * NS
K = 128
EPAD = 163840
EW = EPAD // NW
NDCHUNK = EW // K
CH0 = 18
CH1 = 62
EW0 = CH0 * K
EW1 = CH1 * K
CHMAX = max(CH0, CH1)
RPS = N // NS
NACC = N + 16
ZR = 25
BM = 1000
NB = N // BM

_mesh = plsc.VectorSubcoreMesh(core_axis_name="c", subcore_axis_name="s")


def _deg_body(dst_hbm, out_hbm, idx_d, ones_v, zbuf, accd, sem):
    c = lax.axis_index("c")
    s = lax.axis_index("s")
    w = s * NC + c
    ebase = w * EW

    def _init(i, _):
        ones_v[i, :] = jnp.ones((16,), jnp.float32)
        zbuf[i, :] = jnp.zeros((16,), jnp.float32)
        return 0
    lax.fori_loop(0, K, _init, 0)

    for k in range(5):
        pltpu.sync_copy(zbuf.at[pl.ds(0, 125)],
                        accd.at[pl.ds(s * RPS + k * 125, 125)])

    @pl.when(s == 0)
    def _():
        pltpu.sync_copy(zbuf.at[pl.ds(0, 16)], accd.at[pl.ds(N, 16)])

    plsc.subcore_barrier()

    def _chunk(i, _):
        pltpu.sync_copy(dst_hbm.at[pl.ds(ebase + i * K, K)], idx_d)
        pltpu.sync_copy(ones_v, accd.at[idx_d], add=True)
        return 0
    lax.fori_loop(0, NDCHUNK, _chunk, 0)

    plsc.subcore_barrier()
    pltpu.sync_copy(accd.at[pl.ds(s * RPS, RPS)], out_hbm.at[c, s])


_deg = functools.partial(
    pl.kernel,
    mesh=_mesh,
    out_type=jax.ShapeDtypeStruct((NC, NS, RPS, 16), jnp.float32),
    scratch_types=[
        pltpu.VMEM((K,), jnp.int32),
        pltpu.VMEM((K, 16), jnp.float32),
        pltpu.VMEM((K, 16), jnp.float32),
        pltpu.VMEM_SHARED((NACC, 16), jnp.float32),
        pltpu.SemaphoreType.DMA,
    ],
)(_deg_body)


def _prop_body(g_hbm, src_hbm, dst_hbm, out_hbm,
               src2d, dst2d, rows_a, rows_b, acc, sem_a, sem_b):
    c = lax.axis_index("c")
    s = lax.axis_index("s")
    nsc = jnp.where(c == 0, CH0, CH1)
    ebase = jnp.where(c == 0, s * EW0, NS * EW0 + s * EW1)
    bufs = (rows_a, rows_b)
    sems = (sem_a, sem_b)

    def _prefetch(i, _):
        pltpu.sync_copy(src_hbm.at[pl.ds(ebase + i * K, K)], src2d.at[i])
        pltpu.sync_copy(dst_hbm.at[pl.ds(ebase + i * K, K)], dst2d.at[i])
        return 0
    lax.fori_loop(0, nsc, _prefetch, 0)

    def _zero_own_rows():
        def _initz(i, _):
            for j8 in range(W // 16):
                rows_a[i, pl.ds(j8 * 16, 16)] = jnp.zeros((16,),
                                                          jnp.float32)
            return 0
        lax.fori_loop(0, ZR, _initz, 0)
        for k in range(RPS // ZR):
            pltpu.async_copy(rows_a.at[pl.ds(0, ZR)],
                             acc.at[pl.ds(s * RPS + k * ZR, ZR)], sem_a)
        for k in range(RPS // ZR):
            pltpu.make_async_copy(rows_a.at[pl.ds(0, ZR)],
                                  acc.at[pl.ds(s * RPS, ZR)], sem_a).wait()

    _zero_own_rows()

    @pl.when(s == 0)
    def _():
        pltpu.sync_copy(rows_a.at[pl.ds(0, 16)], acc.at[pl.ds(N, 16)])

    plsc.subcore_barrier()

    def _fire(kk, j):
        pltpu.async_copy(g_hbm.at[src2d.at[kk]], bufs[j], sems[j])

    def _drain(kk, j):
        pltpu.make_async_copy(g_hbm.at[src2d.at[kk]], bufs[j],
                              sems[j]).wait()

    def _frame(p, _):
        @pl.when(p > 0)
        def _():
            def _oadd(i, _):
                for j in range(K // 16):
                    sl = pl.ds(j * 16, 16)
                    src2d[i, sl] = src2d[i, sl] + N
                return 0
            lax.fori_loop(0, nsc, _oadd, 0)

        _fire(0, 0)
        _fire(1, 1)

        def _step(q, _):
            for j in range(2):
                kk = 2 * q + j
                _drain(kk, j)
                pltpu.sync_copy(bufs[j], acc.at[dst2d.at[kk]], add=True)

                @pl.when(kk + 2 < nsc)
                def _():
                    _fire(kk + 2, j)
            return 0
        lax.fori_loop(0, nsc // 2, _step, 0)

        plsc.subcore_barrier()
        pltpu.sync_copy(acc.at[pl.ds(s * RPS, RPS)], out_hbm.at[p, c, s])
        _zero_own_rows()
        plsc.subcore_barrier()
        return 0
    lax.fori_loop(0, P, _frame, 0)


_prop = functools.partial(
    pl.kernel,
    mesh=_mesh,
    out_type=jax.ShapeDtypeStruct((P, NC, NS, RPS, W), jnp.float32),
    scratch_types=[
        pltpu.VMEM((CHMAX, K), jnp.int32),
        pltpu.VMEM((CHMAX, K), jnp.int32),
        pltpu.VMEM((K, W), jnp.float32),
        pltpu.VMEM((K, W), jnp.float32),
        pltpu.VMEM_SHARED((NACC, W), jnp.float32),
        pltpu.SemaphoreType.DMA,
        pltpu.SemaphoreType.DMA,
    ],
)(_prop_body)


def _dinv_body(degp_ref, o_ref):
    deg = degp_ref[0, :, 0] + degp_ref[1, :, 0] + 1.0
    o_ref[...] = jnp.broadcast_to(lax.rsqrt(deg)[:, None], (BM, W))


def _dinv_call(degp):
    return pl.pallas_call(
        _dinv_body,
        grid=(NB,),
        in_specs=[pl.BlockSpec((NC, BM, 16), lambda j: (0, j, 0))],
        out_specs=pl.BlockSpec((BM, W), lambda j: (j, 0)),
        out_shape=jax.ShapeDtypeStruct((N, W), jnp.float32),
    )(degp)


def _stage_a_body(x_ref, w1_ref, dinv_ref, o_ref):
    h0 = jnp.dot(x_ref[0, 0], w1_ref[...], preferred_element_type=jnp.float32)
    h1 = jnp.dot(x_ref[0, 1], w1_ref[...], preferred_element_type=jnp.float32)
    o_ref[0] = jnp.concatenate([h0, h1], axis=1) * dinv_ref[...]


def _stage_a_call(xp, W1, dinvb):
    return pl.pallas_call(
        _stage_a_body,
        grid=(P, NB),
        in_specs=[
            pl.BlockSpec((1, 2, BM, F_IN), lambda p, j: (p, 0, j, 0)),
            pl.BlockSpec((F_IN, H1), lambda p, j: (0, 0)),
            pl.BlockSpec((BM, W), lambda p, j: (j, 0)),
        ],
        out_specs=pl.BlockSpec((1, BM, W), lambda p, j: (p, j, 0)),
        out_shape=jax.ShapeDtypeStruct((P, N, W), jnp.float32),
    )(xp, W1, dinvb)


def _stage_c_body(p_ref, g_ref, dinv_ref, b1_ref, o_ref):
    dinv = dinv_ref[...]
    ssum = p_ref[0, 0] + p_ref[0, 1] + g_ref[0]
    h = jnp.maximum(dinv * ssum + b1_ref[...], 0.0)
    o_ref[0] = dinv * h


def _stage_c_call(p1, g1, dinvb, b1p):
    return pl.pallas_call(
        _stage_c_body,
        grid=(P, NB),
        in_specs=[
            pl.BlockSpec((1, NC, BM, W), lambda p, j: (p, 0, j, 0)),
            pl.BlockSpec((1, BM, W), lambda p, j: (p, j, 0)),
            pl.BlockSpec((BM, W), lambda p, j: (j, 0)),
            pl.BlockSpec((1, W), lambda p, j: (0, 0)),
        ],
        out_specs=pl.BlockSpec((1, BM, W), lambda p, j: (p, j, 0)),
        out_shape=jax.ShapeDtypeStruct((P, N, W), jnp.float32),
    )(p1, g1, dinvb, b1p)


def _stage_e_body(p_ref, g_ref, dinv_ref, w2_ref, b2_ref, o_ref):
    j = pl.program_id(1)
    s2 = dinv_ref[...] * (p_ref[0, 0] + p_ref[0, 1] + g_ref[0])
    ha = jnp.dot(s2[:, :H1], w2_ref[...], preferred_element_type=jnp.float32)
    hb = jnp.dot(s2[:, H1:], w2_ref[...], preferred_element_type=jnp.float32)
    ha = jnp.maximum(ha + b2_ref[...], 0.0)
    hb = jnp.maximum(hb + b2_ref[...], 0.0)
    part = jnp.concatenate([jnp.sum(ha, axis=0, keepdims=True),
                            jnp.sum(hb, axis=0, keepdims=True)], axis=0)

    @pl.when(j == 0)
    def _():
        o_ref[0] = part

    @pl.when(j > 0)
    def _():
        o_ref[0] += part


def _stage_e_call(p2, g2, dinvb, W2, b2r):
    return pl.pallas_call(
        _stage_e_body,
        grid=(P, NB),
        in_specs=[
            pl.BlockSpec((1, NC, BM, W), lambda p, j: (p, 0, j, 0)),
            pl.BlockSpec((1, BM, W), lambda p, j: (p, j, 0)),
            pl.BlockSpec((BM, W), lambda p, j: (j, 0)),
            pl.BlockSpec((H1, H2), lambda p, j: (0, 0)),
            pl.BlockSpec((1, H2), lambda p, j: (0, 0)),
        ],
        out_specs=pl.BlockSpec((1, 2, H2), lambda p, j: (p, 0, 0)),
        out_shape=jax.ShapeDtypeStruct((P, 2, H2), jnp.float32),
    )(p2, g2, dinvb, W2, b2r)


def _lstm_body(es_ref, wih_ref, whh_ref, b_ref, wf_ref, bf_ref, o_ref):
    emb = es_ref[...] * (1.0 / N)
    gx = lax.dot_general(emb, wih_ref[...], (((1,), (1,)), ((), ())),
                         preferred_element_type=jnp.float32)
    b = b_ref[...]
    h = jnp.zeros((1, LH), jnp.float32)
    c = jnp.zeros((1, LH), jnp.float32)
    for t in range(T):
        gh = lax.dot_general(h, whh_ref[...], (((1,), (1,)), ((), ())),
                             preferred_element_type=jnp.float32)
        gates = gx[t:t + 1] + gh + b
        ig = jax.nn.sigmoid(gates[:, 0:LH])
        fg = jax.nn.sigmoid(gates[:, LH:2 * LH])
        gg = jnp.tanh(gates[:, 2 * LH:3 * LH])
        og = jax.nn.sigmoid(gates[:, 3 * LH:4 * LH])
        c = fg * c + ig * gg
        h = og * jnp.tanh(c)
    out = lax.dot_general(h, wf_ref[...], (((1,), (1,)), ((), ())),
                          preferred_element_type=jnp.float32) + bf_ref[...]
    o_ref[...] = out


def _lstm_call(embsum, W_ih, W_hh, bsum, Wf, bfr):
    return pl.pallas_call(
        _lstm_body,
        out_shape=jax.ShapeDtypeStruct((1, A), jnp.float32),
    )(embsum, W_ih, W_hh, bsum, Wf, bfr)


def kernel(x, edge_index, W1, b1, W2, b2, W_ih, W_hh, b_ih, b_hh, Wf, bf):
    src = edge_index[0]
    dst = edge_index[1]
    srcp = jnp.concatenate([src, jnp.zeros((EPAD - E,), jnp.int32)])
    dstp = jnp.concatenate([dst, jnp.full((EPAD - E,), N, jnp.int32)])

    degp = _deg(dstp).reshape(NC, N, 16)
    dinvb = _dinv_call(degp)
    xp = x.reshape(P, 2, N, F_IN)
    g1 = _stage_a_call(xp, W1, dinvb)
    p1 = _prop(g1.reshape(P * N, W), srcp, dstp).reshape(P, NC, N, W)
    b1p = jnp.concatenate([b1, b1]).reshape(1, W)
    g2 = _stage_c_call(p1, g1, dinvb, b1p)
    p2 = _prop(g2.reshape(P * N, W), srcp, dstp).reshape(P, NC, N, W)
    embsum = _stage_e_call(p2, g2, dinvb, W2,
                           b2.reshape(1, H2)).reshape(T, H2)
    return _lstm_call(embsum, W_ih, W_hh,
                      (b_ih + b_hh).reshape(1, 4 * LH), Wf,
                      bf.reshape(1, A))


# speedup vs baseline: 1.4394x; 1.4394x over previous
import functools

import jax
import jax.numpy as jnp
from jax import lax
from jax.experimental import pallas as pl
from jax.experimental.pallas import tpu as pltpu
from jax.experimental.pallas import tpu_sc as plsc

T, N, F_IN = 16, 10000, 256
H1, H2, LH, A = 64, 512, 512, 4
E = 160000

P = T // 2
W = 2 * H1
NC, NS = 2, 16
NW = NC * NS
K = 128
EPAD = 163840
EW = EPAD // NW
NDCHUNK = EW // K
PAIRS0 = 5
ECS = EPAD // NS
NCH = ECS // K
RPS = N // NS
NACC = N + 16
ZR = 25
BM = 1000
NB = N // BM

_mesh = plsc.VectorSubcoreMesh(core_axis_name="c", subcore_axis_name="s")


def _deg_body(dst_hbm, out_hbm, idx_d, ones_v, zbuf, accd, sem):
    c = lax.axis_index("c")
    s = lax.axis_index("s")
    w = s * NC + c
    ebase = w * EW

    def _init(i, _):
        ones_v[i, :] = jnp.ones((16,), jnp.float32)
        zbuf[i, :] = jnp.zeros((16,), jnp.float32)
        return 0
    lax.fori_loop(0, K, _init, 0)

    for k in range(5):
        pltpu.sync_copy(zbuf.at[pl.ds(0, 125)],
                        accd.at[pl.ds(s * RPS + k * 125, 125)])

    @pl.when(s == 0)
    def _():
        pltpu.sync_copy(zbuf.at[pl.ds(0, 16)], accd.at[pl.ds(N, 16)])

    plsc.subcore_barrier()

    def _chunk(i, _):
        pltpu.sync_copy(dst_hbm.at[pl.ds(ebase + i * K, K)], idx_d)
        pltpu.sync_copy(ones_v, accd.at[idx_d], add=True)
        return 0
    lax.fori_loop(0, NDCHUNK, _chunk, 0)

    plsc.subcore_barrier()
    pltpu.sync_copy(accd.at[pl.ds(s * RPS, RPS)], out_hbm.at[c, s])


_deg = functools.partial(
    pl.kernel,
    mesh=_mesh,
    out_type=jax.ShapeDtypeStruct((NC, NS, RPS, 16), jnp.float32),
    scratch_types=[
        pltpu.VMEM((K,), jnp.int32),
        pltpu.VMEM((K, 16), jnp.float32),
        pltpu.VMEM((K, 16), jnp.float32),
        pltpu.VMEM_SHARED((NACC, 16), jnp.float32),
        pltpu.SemaphoreType.DMA,
    ],
)(_deg_body)


def _prop_body(g_hbm, src_hbm, dst_hbm, out_hbm,
               si0, si1, di0, di1, rows_a, rows_b, acc,
               isem0, isem1, gsem0, gsem1):
    c = lax.axis_index("c")
    s = lax.axis_index("s")
    ebase = s * ECS
    sbufs = (si0, si1)
    dbufs = (di0, di1)
    isems = (isem0, isem1)
    gsems = (gsem0, gsem1)
    rbufs = (rows_a, rows_b)
    npair = jnp.where(c == 0, PAIRS0, P - PAIRS0)
    pbase = jnp.where(c == 0, 0, PAIRS0)

    def _zero_own_rows():
        def _initz(i, _):
            for j8 in range(W // 16):
                rows_a[i, pl.ds(j8 * 16, 16)] = jnp.zeros((16,),
                                                          jnp.float32)
            return 0
        lax.fori_loop(0, ZR, _initz, 0)
        for k in range(RPS // ZR):
            pltpu.async_copy(rows_a.at[pl.ds(0, ZR)],
                             acc.at[pl.ds(s * RPS + k * ZR, ZR)], gsem0)
        for k in range(RPS // ZR):
            pltpu.make_async_copy(rows_a.at[pl.ds(0, ZR)],
                                  acc.at[pl.ds(s * RPS, ZR)], gsem0).wait()

    _zero_own_rows()

    @pl.when(s == 0)
    def _():
        pltpu.sync_copy(rows_a.at[pl.ds(0, 16)], acc.at[pl.ds(N, 16)])

    plsc.subcore_barrier()

    def _fire_idx(kk, j):
        pltpu.async_copy(src_hbm.at[pl.ds(ebase + kk * K, K)], sbufs[j],
                         isems[j])
        pltpu.async_copy(dst_hbm.at[pl.ds(ebase + kk * K, K)], dbufs[j],
                         isems[j])

    def _wait_idx(j):
        pltpu.make_async_copy(src_hbm.at[pl.ds(ebase, K)], sbufs[j],
                              isems[j]).wait()
        pltpu.make_async_copy(dst_hbm.at[pl.ds(ebase, K)], dbufs[j],
                              isems[j]).wait()

    def _oadd(j, off):
        for m in range(K // 16):
            sl = pl.ds(m * 16, 16)
            sbufs[j][sl] = sbufs[j][sl] + off

    def _frame(q, _):
        p = pbase + q
        off = p * N
        _fire_idx(0, 0)
        _fire_idx(1, 1)
        _wait_idx(0)
        _oadd(0, off)
        pltpu.async_copy(g_hbm.at[si0], rows_a, gsem0)

        def _step(q2, _):
            for j in range(2):
                kk = 2 * q2 + j
                jn = 1 - j

                @pl.when(kk + 1 < NCH)
                def _():
                    _wait_idx(jn)
                    _oadd(jn, off)
                    pltpu.async_copy(g_hbm.at[sbufs[jn]], rbufs[jn],
                                     gsems[jn])

                pltpu.make_async_copy(g_hbm.at[sbufs[j]], rbufs[j],
                                      gsems[j]).wait()
                pltpu.sync_copy(rbufs[j], acc.at[dbufs[j]], add=True)

                @pl.when(kk + 2 < NCH)
                def _():
                    _fire_idx(kk + 2, j)
            return 0
        lax.fori_loop(0, NCH // 2, _step, 0)

        plsc.subcore_barrier()
        pltpu.sync_copy(acc.at[pl.ds(s * RPS, RPS)], out_hbm.at[p, s])
        _zero_own_rows()
        plsc.subcore_barrier()
        return 0
    lax.fori_loop(0, npair, _frame, 0)


_prop = functools.partial(
    pl.kernel,
    mesh=_mesh,
    out_type=jax.ShapeDtypeStruct((P, NS, RPS, W), jnp.float32),
    scratch_types=[
        pltpu.VMEM((K,), jnp.int32),
        pltpu.VMEM((K,), jnp.int32),
        pltpu.VMEM((K,), jnp.int32),
        pltpu.VMEM((K,), jnp.int32),
        pltpu.VMEM((K, W), jnp.float32),
        pltpu.VMEM((K, W), jnp.float32),
        pltpu.VMEM_SHARED((NACC, W), jnp.float32),
        pltpu.SemaphoreType.DMA,
        pltpu.SemaphoreType.DMA,
        pltpu.SemaphoreType.DMA,
        pltpu.SemaphoreType.DMA,
    ],
)(_prop_body)


def _dinv_body(degp_ref, o_ref):
    deg = degp_ref[0, :, 0] + degp_ref[1, :, 0] + 1.0
    o_ref[...] = jnp.broadcast_to(lax.rsqrt(deg)[:, None], (BM, W))


def _dinv_call(degp):
    return pl.pallas_call(
        _dinv_body,
        grid=(NB,),
        in_specs=[pl.BlockSpec((NC, BM, 16), lambda j: (0, j, 0))],
        out_specs=pl.BlockSpec((BM, W), lambda j: (j, 0)),
        out_shape=jax.ShapeDtypeStruct((N, W), jnp.float32),
    )(degp)


def _stage_a_body(x_ref, w1_ref, dinv_ref, o_ref):
    h0 = jnp.dot(x_ref[0, 0], w1_ref[...], preferred_element_type=jnp.float32)
    h1 = jnp.dot(x_ref[0, 1], w1_ref[...], preferred_element_type=jnp.float32)
    o_ref[0] = jnp.concatenate([h0, h1], axis=1) * dinv_ref[...]


def _stage_a_call(xp, W1, dinvb):
    return pl.pallas_call(
        _stage_a_body,
        grid=(P, NB),
        in_specs=[
            pl.BlockSpec((1, 2, BM, F_IN), lambda p, j: (p, 0, j, 0)),
            pl.BlockSpec((F_IN, H1), lambda p, j: (0, 0)),
            pl.BlockSpec((BM, W), lambda p, j: (j, 0)),
        ],
        out_specs=pl.BlockSpec((1, BM, W), lambda p, j: (p, j, 0)),
        out_shape=jax.ShapeDtypeStruct((P, N, W), jnp.float32),
    )(xp, W1, dinvb)


def _stage_c_body(p_ref, g_ref, dinv_ref, b1_ref, o_ref):
    dinv = dinv_ref[...]
    ssum = p_ref[0] + g_ref[0]
    h = jnp.maximum(dinv * ssum + b1_ref[...], 0.0)
    o_ref[0] = dinv * h


def _stage_c_call(p1, g1, dinvb, b1p):
    return pl.pallas_call(
        _stage_c_body,
        grid=(P, NB),
        in_specs=[
            pl.BlockSpec((1, BM, W), lambda p, j: (p, j, 0)),
            pl.BlockSpec((1, BM, W), lambda p, j: (p, j, 0)),
            pl.BlockSpec((BM, W), lambda p, j: (j, 0)),
            pl.BlockSpec((1, W), lambda p, j: (0, 0)),
        ],
        out_specs=pl.BlockSpec((1, BM, W), lambda p, j: (p, j, 0)),
        out_shape=jax.ShapeDtypeStruct((P, N, W), jnp.float32),
    )(p1, g1, dinvb, b1p)


def _stage_e_body(p_ref, g_ref, dinv_ref, w2_ref, b2_ref, o_ref):
    j = pl.program_id(1)
    s2 = dinv_ref[...] * (p_ref[0] + g_ref[0])
    ha = jnp.dot(s2[:, :H1], w2_ref[...], preferred_element_type=jnp.float32)
    hb = jnp.dot(s2[:, H1:], w2_ref[...], preferred_element_type=jnp.float32)
    ha = jnp.maximum(ha + b2_ref[...], 0.0)
    hb = jnp.maximum(hb + b2_ref[...], 0.0)
    part = jnp.concatenate([jnp.sum(ha, axis=0, keepdims=True),
                            jnp.sum(hb, axis=0, keepdims=True)], axis=0)

    @pl.when(j == 0)
    def _():
        o_ref[0] = part

    @pl.when(j > 0)
    def _():
        o_ref[0] += part


def _stage_e_call(p2, g2, dinvb, W2, b2r):
    return pl.pallas_call(
        _stage_e_body,
        grid=(P, NB),
        in_specs=[
            pl.BlockSpec((1, BM, W), lambda p, j: (p, j, 0)),
            pl.BlockSpec((1, BM, W), lambda p, j: (p, j, 0)),
            pl.BlockSpec((BM, W), lambda p, j: (j, 0)),
            pl.BlockSpec((H1, H2), lambda p, j: (0, 0)),
            pl.BlockSpec((1, H2), lambda p, j: (0, 0)),
        ],
        out_specs=pl.BlockSpec((1, 2, H2), lambda p, j: (p, 0, 0)),
        out_shape=jax.ShapeDtypeStruct((P, 2, H2), jnp.float32),
    )(p2, g2, dinvb, W2, b2r)


def _lstm_body(es_ref, wih_ref, whh_ref, b_ref, wf_ref, bf_ref, o_ref):
    emb = es_ref[...] * (1.0 / N)
    gx = lax.dot_general(emb, wih_ref[...], (((1,), (1,)), ((), ())),
                         preferred_element_type=jnp.float32)
    b = b_ref[...]
    h = jnp.zeros((1, LH), jnp.float32)
    c = jnp.zeros((1, LH), jnp.float32)
    for t in range(T):
        gh = lax.dot_general(h, whh_ref[...], (((1,), (1,)), ((), ())),
                             preferred_element_type=jnp.float32)
        gates = gx[t:t + 1] + gh + b
        ig = jax.nn.sigmoid(gates[:, 0:LH])
        fg = jax.nn.sigmoid(gates[:, LH:2 * LH])
        gg = jnp.tanh(gates[:, 2 * LH:3 * LH])
        og = jax.nn.sigmoid(gates[:, 3 * LH:4 * LH])
        c = fg * c + ig * gg
        h = og * jnp.tanh(c)
    out = lax.dot_general(h, wf_ref[...], (((1,), (1,)), ((), ())),
                          preferred_element_type=jnp.float32) + bf_ref[...]
    o_ref[...] = out


def _lstm_call(embsum, W_ih, W_hh, bsum, Wf, bfr):
    return pl.pallas_call(
        _lstm_body,
        out_shape=jax.ShapeDtypeStruct((1, A), jnp.float32),
    )(embsum, W_ih, W_hh, bsum, Wf, bfr)


def kernel(x, edge_index, W1, b1, W2, b2, W_ih, W_hh, b_ih, b_hh, Wf, bf):
    src = edge_index[0]
    dst = edge_index[1]
    srcp = jnp.concatenate([src, jnp.zeros((EPAD - E,), jnp.int32)])
    dstp = jnp.concatenate([dst, jnp.full((EPAD - E,), N, jnp.int32)])

    degp = _deg(dstp).reshape(NC, N, 16)
    dinvb = _dinv_call(degp)
    xp = x.reshape(P, 2, N, F_IN)
    g1 = _stage_a_call(xp, W1, dinvb)
    p1 = _prop(g1.reshape(P * N, W), srcp, dstp).reshape(P, N, W)
    b1p = jnp.concatenate([b1, b1]).reshape(1, W)
    g2 = _stage_c_call(p1, g1, dinvb, b1p)
    p2 = _prop(g2.reshape(P * N, W), srcp, dstp).reshape(P, N, W)
    embsum = _stage_e_call(p2, g2, dinvb, W2,
                           b2.reshape(1, H2)).reshape(T, H2)
    return _lstm_call(embsum, W_ih, W_hh,
                      (b_ih + b_hh).reshape(1, 4 * LH), Wf,
                      bf.reshape(1, A))


# speedup vs baseline: 1.6737x; 1.1628x over previous
import functools

import jax
import jax.numpy as jnp
from jax import lax
from jax.experimental import pallas as pl
from jax.experimental.pallas import tpu as pltpu
from jax.experimental.pallas import tpu_sc as plsc

T, N, F_IN = 16, 10000, 256
H1, H2, LH, A = 64, 512, 512, 4
E = 160000

P = T // 2
W = 2 * H1
NC, NS = 2, 16
NW = NC * NS
K = 128
EPAD = 163840
EW = EPAD // NW
NDCHUNK = EW // K
PAIRS0 = 4
ECS = EPAD // NS
NCH = ECS // K
RPS = N // NS
NACC = N + 16
ZR = 25
BM = 1000
NB = N // BM

_mesh = plsc.VectorSubcoreMesh(core_axis_name="c", subcore_axis_name="s")


def _deg_body(dst_hbm, out_hbm, idx_d, ones_v, zbuf, accd, sem):
    c = lax.axis_index("c")
    s = lax.axis_index("s")
    w = s * NC + c
    ebase = w * EW

    def _init(i, _):
        ones_v[i, :] = jnp.ones((16,), jnp.float32)
        zbuf[i, :] = jnp.zeros((16,), jnp.float32)
        return 0
    lax.fori_loop(0, K, _init, 0)

    for k in range(5):
        pltpu.sync_copy(zbuf.at[pl.ds(0, 125)],
                        accd.at[pl.ds(s * RPS + k * 125, 125)])

    @pl.when(s == 0)
    def _():
        pltpu.sync_copy(zbuf.at[pl.ds(0, 16)], accd.at[pl.ds(N, 16)])

    plsc.subcore_barrier()

    def _chunk(i, _):
        pltpu.sync_copy(dst_hbm.at[pl.ds(ebase + i * K, K)], idx_d)
        pltpu.sync_copy(ones_v, accd.at[idx_d], add=True)
        return 0
    lax.fori_loop(0, NDCHUNK, _chunk, 0)

    plsc.subcore_barrier()
    pltpu.sync_copy(accd.at[pl.ds(s * RPS, RPS)], out_hbm.at[c, s])


_deg = functools.partial(
    pl.kernel,
    mesh=_mesh,
    out_type=jax.ShapeDtypeStruct((NC, NS, RPS, 16), jnp.float32),
    scratch_types=[
        pltpu.VMEM((K,), jnp.int32),
        pltpu.VMEM((K, 16), jnp.float32),
        pltpu.VMEM((K, 16), jnp.float32),
        pltpu.VMEM_SHARED((NACC, 16), jnp.float32),
        pltpu.SemaphoreType.DMA,
    ],
)(_deg_body)


def _prop_body(g_hbm, src_hbm, dst_hbm, out_hbm,
               si0, si1, di0, di1, rows_a, rows_b, acc,
               isem0, isem1, gsem0, gsem1):
    c = lax.axis_index("c")
    s = lax.axis_index("s")
    ebase = s * ECS
    sbufs = (si0, si1)
    dbufs = (di0, di1)
    isems = (isem0, isem1)
    gsems = (gsem0, gsem1)
    rbufs = (rows_a, rows_b)
    npair = jnp.where(c == 0, PAIRS0, P - PAIRS0)
    pbase = jnp.where(c == 0, 0, PAIRS0)

    def _zero_own_rows():
        def _initz(i, _):
            for j8 in range(W // 16):
                rows_a[i, pl.ds(j8 * 16, 16)] = jnp.zeros((16,),
                                                          jnp.float32)
            return 0
        lax.fori_loop(0, ZR, _initz, 0)
        for k in range(RPS // ZR):
            pltpu.async_copy(rows_a.at[pl.ds(0, ZR)],
                             acc.at[pl.ds(s * RPS + k * ZR, ZR)], gsem0)
        for k in range(RPS // ZR):
            pltpu.make_async_copy(rows_a.at[pl.ds(0, ZR)],
                                  acc.at[pl.ds(s * RPS, ZR)], gsem0).wait()

    _zero_own_rows()

    @pl.when(s == 0)
    def _():
        pltpu.sync_copy(rows_a.at[pl.ds(0, 16)], acc.at[pl.ds(N, 16)])

    plsc.subcore_barrier()

    def _fire_idx(kk, j):
        pltpu.async_copy(src_hbm.at[pl.ds(ebase + kk * K, K)], sbufs[j],
                         isems[j])
        pltpu.async_copy(dst_hbm.at[pl.ds(ebase + kk * K, K)], dbufs[j],
                         isems[j])

    def _wait_idx(j):
        pltpu.make_async_copy(src_hbm.at[pl.ds(ebase, K)], sbufs[j],
                              isems[j]).wait()
        pltpu.make_async_copy(dst_hbm.at[pl.ds(ebase, K)], dbufs[j],
                              isems[j]).wait()

    def _oadd(j, off):
        for m in range(K // 16):
            sl = pl.ds(m * 16, 16)
            sbufs[j][sl] = sbufs[j][sl] + off

    def _frame(q, _):
        p = pbase + q
        off = p * N
        _fire_idx(0, 0)
        _fire_idx(1, 1)
        _wait_idx(0)
        _oadd(0, off)
        pltpu.async_copy(g_hbm.at[si0], rows_a, gsem0)

        def _step(q2, _):
            for j in range(2):
                kk = 2 * q2 + j
                jn = 1 - j

                @pl.when(kk + 1 < NCH)
                def _():
                    _wait_idx(jn)
                    _oadd(jn, off)
                    pltpu.async_copy(g_hbm.at[sbufs[jn]], rbufs[jn],
                                     gsems[jn])

                pltpu.make_async_copy(g_hbm.at[sbufs[j]], rbufs[j],
                                      gsems[j]).wait()
                pltpu.sync_copy(rbufs[j], acc.at[dbufs[j]], add=True)

                @pl.when(kk + 2 < NCH)
                def _():
                    _fire_idx(kk + 2, j)
            return 0
        lax.fori_loop(0, NCH // 2, _step, 0)

        plsc.subcore_barrier()
        pltpu.sync_copy(acc.at[pl.ds(s * RPS, RPS)], out_hbm.at[p, s])
        _zero_own_rows()
        plsc.subcore_barrier()
        return 0
    lax.fori_loop(0, npair, _frame, 0)


_prop = functools.partial(
    pl.kernel,
    mesh=_mesh,
    out_type=jax.ShapeDtypeStruct((P, NS, RPS, W), jnp.float32),
    scratch_types=[
        pltpu.VMEM((K,), jnp.int32),
        pltpu.VMEM((K,), jnp.int32),
        pltpu.VMEM((K,), jnp.int32),
        pltpu.VMEM((K,), jnp.int32),
        pltpu.VMEM((K, W), jnp.float32),
        pltpu.VMEM((K, W), jnp.float32),
        pltpu.VMEM_SHARED((NACC, W), jnp.float32),
        pltpu.SemaphoreType.DMA,
        pltpu.SemaphoreType.DMA,
        pltpu.SemaphoreType.DMA,
        pltpu.SemaphoreType.DMA,
    ],
)(_prop_body)


def _dinv_body(degp_ref, o_ref):
    deg = degp_ref[0, :, 0] + degp_ref[1, :, 0] + 1.0
    o_ref[...] = jnp.broadcast_to(lax.rsqrt(deg)[:, None], (BM, W))


def _dinv_call(degp):
    return pl.pallas_call(
        _dinv_body,
        grid=(NB,),
        in_specs=[pl.BlockSpec((NC, BM, 16), lambda j: (0, j, 0))],
        out_specs=pl.BlockSpec((BM, W), lambda j: (j, 0)),
        out_shape=jax.ShapeDtypeStruct((N, W), jnp.float32),
    )(degp)


def _stage_a_body(x_ref, w1_ref, dinv_ref, o_ref):
    h0 = jnp.dot(x_ref[0, 0], w1_ref[...], preferred_element_type=jnp.float32)
    h1 = jnp.dot(x_ref[0, 1], w1_ref[...], preferred_element_type=jnp.float32)
    o_ref[0] = jnp.concatenate([h0, h1], axis=1) * dinv_ref[...]


def _stage_a_call(xp, W1, dinvb):
    return pl.pallas_call(
        _stage_a_body,
        grid=(P, NB),
        in_specs=[
            pl.BlockSpec((1, 2, BM, F_IN), lambda p, j: (p, 0, j, 0)),
            pl.BlockSpec((F_IN, H1), lambda p, j: (0, 0)),
            pl.BlockSpec((BM, W), lambda p, j: (j, 0)),
        ],
        out_specs=pl.BlockSpec((1, BM, W), lambda p, j: (p, j, 0)),
        out_shape=jax.ShapeDtypeStruct((P, N, W), jnp.float32),
    )(xp, W1, dinvb)


def _stage_c_body(p_ref, g_ref, dinv_ref, b1_ref, o_ref):
    dinv = dinv_ref[...]
    ssum = p_ref[0] + g_ref[0]
    h = jnp.maximum(dinv * ssum + b1_ref[...], 0.0)
    o_ref[0] = dinv * h


def _stage_c_call(p1, g1, dinvb, b1p):
    return pl.pallas_call(
        _stage_c_body,
        grid=(P, NB),
        in_specs=[
            pl.BlockSpec((1, BM, W), lambda p, j: (p, j, 0)),
            pl.BlockSpec((1, BM, W), lambda p, j: (p, j, 0)),
            pl.BlockSpec((BM, W), lambda p, j: (j, 0)),
            pl.BlockSpec((1, W), lambda p, j: (0, 0)),
        ],
        out_specs=pl.BlockSpec((1, BM, W), lambda p, j: (p, j, 0)),
        out_shape=jax.ShapeDtypeStruct((P, N, W), jnp.float32),
    )(p1, g1, dinvb, b1p)


def _stage_e_body(p_ref, g_ref, dinv_ref, w2_ref, b2_ref, o_ref):
    j = pl.program_id(1)
    s2 = dinv_ref[...] * (p_ref[0] + g_ref[0])
    ha = jnp.dot(s2[:, :H1], w2_ref[...], preferred_element_type=jnp.float32)
    hb = jnp.dot(s2[:, H1:], w2_ref[...], preferred_element_type=jnp.float32)
    ha = jnp.maximum(ha + b2_ref[...], 0.0)
    hb = jnp.maximum(hb + b2_ref[...], 0.0)
    part = jnp.concatenate([jnp.sum(ha, axis=0, keepdims=True),
                            jnp.sum(hb, axis=0, keepdims=True)], axis=0)

    @pl.when(j == 0)
    def _():
        o_ref[0] = part

    @pl.when(j > 0)
    def _():
        o_ref[0] += part


def _stage_e_call(p2, g2, dinvb, W2, b2r):
    return pl.pallas_call(
        _stage_e_body,
        grid=(P, NB),
        in_specs=[
            pl.BlockSpec((1, BM, W), lambda p, j: (p, j, 0)),
            pl.BlockSpec((1, BM, W), lambda p, j: (p, j, 0)),
            pl.BlockSpec((BM, W), lambda p, j: (j, 0)),
            pl.BlockSpec((H1, H2), lambda p, j: (0, 0)),
            pl.BlockSpec((1, H2), lambda p, j: (0, 0)),
        ],
        out_specs=pl.BlockSpec((1, 2, H2), lambda p, j: (p, 0, 0)),
        out_shape=jax.ShapeDtypeStruct((P, 2, H2), jnp.float32),
    )(p2, g2, dinvb, W2, b2r)


def _lstm_body(es_ref, wih_ref, whh_ref, b_ref, wf_ref, bf_ref, o_ref):
    emb = es_ref[...] * (1.0 / N)
    gx = lax.dot_general(emb, wih_ref[...], (((1,), (1,)), ((), ())),
                         preferred_element_type=jnp.float32)
    b = b_ref[...]
    h = jnp.zeros((1, LH), jnp.float32)
    c = jnp.zeros((1, LH), jnp.float32)
    for t in range(T):
        gh = lax.dot_general(h, whh_ref[...], (((1,), (1,)), ((), ())),
                             preferred_element_type=jnp.float32)
        gates = gx[t:t + 1] + gh + b
        ig = jax.nn.sigmoid(gates[:, 0:LH])
        fg = jax.nn.sigmoid(gates[:, LH:2 * LH])
        gg = jnp.tanh(gates[:, 2 * LH:3 * LH])
        og = jax.nn.sigmoid(gates[:, 3 * LH:4 * LH])
        c = fg * c + ig * gg
        h = og * jnp.tanh(c)
    out = lax.dot_general(h, wf_ref[...], (((1,), (1,)), ((), ())),
                          preferred_element_type=jnp.float32) + bf_ref[...]
    o_ref[...] = out


def _lstm_call(embsum, W_ih, W_hh, bsum, Wf, bfr):
    return pl.pallas_call(
        _lstm_body,
        out_shape=jax.ShapeDtypeStruct((1, A), jnp.float32),
    )(embsum, W_ih, W_hh, bsum, Wf, bfr)


def kernel(x, edge_index, W1, b1, W2, b2, W_ih, W_hh, b_ih, b_hh, Wf, bf):
    src = edge_index[0]
    dst = edge_index[1]
    srcp = jnp.concatenate([src, jnp.zeros((EPAD - E,), jnp.int32)])
    dstp = jnp.concatenate([dst, jnp.full((EPAD - E,), N, jnp.int32)])

    degp = _deg(dstp).reshape(NC, N, 16)
    dinvb = _dinv_call(degp)
    xp = x.reshape(P, 2, N, F_IN)
    g1 = _stage_a_call(xp, W1, dinvb)
    p1 = _prop(g1.reshape(P * N, W), srcp, dstp).reshape(P, N, W)
    b1p = jnp.concatenate([b1, b1]).reshape(1, W)
    g2 = _stage_c_call(p1, g1, dinvb, b1p)
    p2 = _prop(g2.reshape(P * N, W), srcp, dstp).reshape(P, N, W)
    embsum = _stage_e_call(p2, g2, dinvb, W2,
                           b2.reshape(1, H2)).reshape(T, H2)
    return _lstm_call(embsum, W_ih, W_hh,
                      (b_ih + b_hh).reshape(1, 4 * LH), Wf,
                      bf.reshape(1, A))


# speedup vs baseline: 1.6811x; 1.0044x over previous
import functools

import jax
import jax.numpy as jnp
from jax import lax
from jax.experimental import pallas as pl
from jax.experimental.pallas import tpu as pltpu
from jax.experimental.pallas import tpu_sc as plsc

T, N, F_IN = 16, 10000, 256
H1, H2, LH, A = 64, 512, 512, 4
E = 160000

P = T // 2
W = 2 * H1
NC, NS = 2, 16
NW = NC * NS
K = 128
EPAD = 163840
EW = EPAD // NW
NDCHUNK = EW // K
PAIRS0 = 4
ECS = EPAD // NS
NCH = ECS // K
RPS = N // NS
NACC = N + 16
ZR = 25
BM = 1000
NB = N // BM

_mesh = plsc.VectorSubcoreMesh(core_axis_name="c", subcore_axis_name="s")


def _deg_body(dst_hbm, out_hbm, idx_d, ones_v, zbuf, accd, sem):
    c = lax.axis_index("c")
    s = lax.axis_index("s")
    w = s * NC + c
    ebase = w * EW

    def _init(i, _):
        ones_v[i, :] = jnp.ones((16,), jnp.float32)
        zbuf[i, :] = jnp.zeros((16,), jnp.float32)
        return 0
    lax.fori_loop(0, K, _init, 0)

    for k in range(5):
        pltpu.sync_copy(zbuf.at[pl.ds(0, 125)],
                        accd.at[pl.ds(s * RPS + k * 125, 125)])

    @pl.when(s == 0)
    def _():
        pltpu.sync_copy(zbuf.at[pl.ds(0, 16)], accd.at[pl.ds(N, 16)])

    plsc.subcore_barrier()

    def _chunk(i, _):
        pltpu.sync_copy(dst_hbm.at[pl.ds(ebase + i * K, K)], idx_d)
        pltpu.sync_copy(ones_v, accd.at[idx_d], add=True)
        return 0
    lax.fori_loop(0, NDCHUNK, _chunk, 0)

    plsc.subcore_barrier()
    pltpu.sync_copy(accd.at[pl.ds(s * RPS, RPS)], out_hbm.at[c, s])


_deg = functools.partial(
    pl.kernel,
    mesh=_mesh,
    out_type=jax.ShapeDtypeStruct((NC, NS, RPS, 16), jnp.float32),
    scratch_types=[
        pltpu.VMEM((K,), jnp.int32),
        pltpu.VMEM((K, 16), jnp.float32),
        pltpu.VMEM((K, 16), jnp.float32),
        pltpu.VMEM_SHARED((NACC, 16), jnp.float32),
        pltpu.SemaphoreType.DMA,
    ],
)(_deg_body)


def _prop_body(g_hbm, src_hbm, dst_hbm, out_hbm,
               si0, si1, di0, di1, rows_a, rows_b, zbuf, acc,
               isem0, isem1, gsem0, gsem1, osem):
    c = lax.axis_index("c")
    s = lax.axis_index("s")
    ebase = s * ECS
    sbufs = (si0, si1)
    dbufs = (di0, di1)
    isems = (isem0, isem1)
    gsems = (gsem0, gsem1)
    rbufs = (rows_a, rows_b)
    npair = jnp.where(c == 0, PAIRS0, P - PAIRS0)
    pbase = jnp.where(c == 0, 0, PAIRS0)

    def _initz(i, _):
        for j8 in range(W // 16):
            zbuf[i, pl.ds(j8 * 16, 16)] = jnp.zeros((16,), jnp.float32)
        return 0
    lax.fori_loop(0, ZR, _initz, 0)

    def _zero_own_rows():
        for k in range(RPS // ZR):
            pltpu.async_copy(zbuf,
                             acc.at[pl.ds(s * RPS + k * ZR, ZR)], osem)
        for k in range(RPS // ZR):
            pltpu.make_async_copy(zbuf,
                                  acc.at[pl.ds(s * RPS, ZR)], osem).wait()

    _zero_own_rows()

    @pl.when(s == 0)
    def _():
        pltpu.sync_copy(zbuf.at[pl.ds(0, 16)], acc.at[pl.ds(N, 16)])

    plsc.subcore_barrier()

    def _fire_idx(kk, j):
        pltpu.async_copy(src_hbm.at[pl.ds(ebase + kk * K, K)], sbufs[j],
                         isems[j])
        pltpu.async_copy(dst_hbm.at[pl.ds(ebase + kk * K, K)], dbufs[j],
                         isems[j])

    def _wait_idx(j):
        pltpu.make_async_copy(src_hbm.at[pl.ds(ebase, K)], sbufs[j],
                              isems[j]).wait()
        pltpu.make_async_copy(dst_hbm.at[pl.ds(ebase, K)], dbufs[j],
                              isems[j]).wait()

    def _oadd(j, off):
        for m in range(K // 16):
            sl = pl.ds(m * 16, 16)
            sbufs[j][sl] = sbufs[j][sl] + off

    def _prime(p):
        _fire_idx(0, 0)
        _fire_idx(1, 1)
        _wait_idx(0)
        _oadd(0, p * N)
        pltpu.async_copy(g_hbm.at[si0], rows_a, gsem0)

    _prime(pbase)

    def _frame(q, _):
        p = pbase + q
        off = p * N

        def _step(q2, _):
            for j in range(2):
                kk = 2 * q2 + j
                jn = 1 - j

                @pl.when(kk + 1 < NCH)
                def _():
                    _wait_idx(jn)
                    _oadd(jn, off)
                    pltpu.async_copy(g_hbm.at[sbufs[jn]], rbufs[jn],
                                     gsems[jn])

                pltpu.make_async_copy(g_hbm.at[sbufs[j]], rbufs[j],
                                      gsems[j]).wait()
                pltpu.sync_copy(rbufs[j], acc.at[dbufs[j]], add=True)

                @pl.when(kk + 2 < NCH)
                def _():
                    _fire_idx(kk + 2, j)
            return 0
        lax.fori_loop(0, NCH // 2, _step, 0)

        plsc.subcore_barrier()

        @pl.when(q + 1 < npair)
        def _():
            _prime(p + 1)

        pltpu.sync_copy(acc.at[pl.ds(s * RPS, RPS)], out_hbm.at[p, s])
        _zero_own_rows()
        plsc.subcore_barrier()
        return 0
    lax.fori_loop(0, npair, _frame, 0)


_prop = functools.partial(
    pl.kernel,
    mesh=_mesh,
    out_type=jax.ShapeDtypeStruct((P, NS, RPS, W), jnp.float32),
    scratch_types=[
        pltpu.VMEM((K,), jnp.int32),
        pltpu.VMEM((K,), jnp.int32),
        pltpu.VMEM((K,), jnp.int32),
        pltpu.VMEM((K,), jnp.int32),
        pltpu.VMEM((K, W), jnp.float32),
        pltpu.VMEM((K, W), jnp.float32),
        pltpu.VMEM((ZR, W), jnp.float32),
        pltpu.VMEM_SHARED((NACC, W), jnp.float32),
        pltpu.SemaphoreType.DMA,
        pltpu.SemaphoreType.DMA,
        pltpu.SemaphoreType.DMA,
        pltpu.SemaphoreType.DMA,
        pltpu.SemaphoreType.DMA,
    ],
)(_prop_body)


def _dinv_body(degp_ref, o_ref):
    deg = degp_ref[0, :, 0] + degp_ref[1, :, 0] + 1.0
    o_ref[...] = jnp.broadcast_to(lax.rsqrt(deg)[:, None], (BM, W))


def _dinv_call(degp):
    return pl.pallas_call(
        _dinv_body,
        grid=(NB,),
        in_specs=[pl.BlockSpec((NC, BM, 16), lambda j: (0, j, 0))],
        out_specs=pl.BlockSpec((BM, W), lambda j: (j, 0)),
        out_shape=jax.ShapeDtypeStruct((N, W), jnp.float32),
    )(degp)


def _stage_a_body(x_ref, w1_ref, dinv_ref, o_ref):
    h0 = jnp.dot(x_ref[0, 0], w1_ref[...], preferred_element_type=jnp.float32)
    h1 = jnp.dot(x_ref[0, 1], w1_ref[...], preferred_element_type=jnp.float32)
    o_ref[0] = jnp.concatenate([h0, h1], axis=1) * dinv_ref[...]


def _stage_a_call(xp, W1, dinvb):
    return pl.pallas_call(
        _stage_a_body,
        grid=(P, NB),
        in_specs=[
            pl.BlockSpec((1, 2, BM, F_IN), lambda p, j: (p, 0, j, 0)),
            pl.BlockSpec((F_IN, H1), lambda p, j: (0, 0)),
            pl.BlockSpec((BM, W), lambda p, j: (j, 0)),
        ],
        out_specs=pl.BlockSpec((1, BM, W), lambda p, j: (p, j, 0)),
        out_shape=jax.ShapeDtypeStruct((P, N, W), jnp.float32),
    )(xp, W1, dinvb)


def _stage_c_body(p_ref, g_ref, dinv_ref, b1_ref, o_ref):
    dinv = dinv_ref[...]
    ssum = p_ref[0] + g_ref[0]
    h = jnp.maximum(dinv * ssum + b1_ref[...], 0.0)
    o_ref[0] = dinv * h


def _stage_c_call(p1, g1, dinvb, b1p):
    return pl.pallas_call(
        _stage_c_body,
        grid=(P, NB),
        in_specs=[
            pl.BlockSpec((1, BM, W), lambda p, j: (p, j, 0)),
            pl.BlockSpec((1, BM, W), lambda p, j: (p, j, 0)),
            pl.BlockSpec((BM, W), lambda p, j: (j, 0)),
            pl.BlockSpec((1, W), lambda p, j: (0, 0)),
        ],
        out_specs=pl.BlockSpec((1, BM, W), lambda p, j: (p, j, 0)),
        out_shape=jax.ShapeDtypeStruct((P, N, W), jnp.float32),
    )(p1, g1, dinvb, b1p)


def _stage_e_body(p_ref, g_ref, dinv_ref, w2_ref, b2_ref, o_ref):
    j = pl.program_id(1)
    s2 = dinv_ref[...] * (p_ref[0] + g_ref[0])
    ha = jnp.dot(s2[:, :H1], w2_ref[...], preferred_element_type=jnp.float32)
    hb = jnp.dot(s2[:, H1:], w2_ref[...], preferred_element_type=jnp.float32)
    ha = jnp.maximum(ha + b2_ref[...], 0.0)
    hb = jnp.maximum(hb + b2_ref[...], 0.0)
    part = jnp.concatenate([jnp.sum(ha, axis=0, keepdims=True),
                            jnp.sum(hb, axis=0, keepdims=True)], axis=0)

    @pl.when(j == 0)
    def _():
        o_ref[0] = part

    @pl.when(j > 0)
    def _():
        o_ref[0] += part


def _stage_e_call(p2, g2, dinvb, W2, b2r):
    return pl.pallas_call(
        _stage_e_body,
        grid=(P, NB),
        in_specs=[
            pl.BlockSpec((1, BM, W), lambda p, j: (p, j, 0)),
            pl.BlockSpec((1, BM, W), lambda p, j: (p, j, 0)),
            pl.BlockSpec((BM, W), lambda p, j: (j, 0)),
            pl.BlockSpec((H1, H2), lambda p, j: (0, 0)),
            pl.BlockSpec((1, H2), lambda p, j: (0, 0)),
        ],
        out_specs=pl.BlockSpec((1, 2, H2), lambda p, j: (p, 0, 0)),
        out_shape=jax.ShapeDtypeStruct((P, 2, H2), jnp.float32),
    )(p2, g2, dinvb, W2, b2r)


def _lstm_body(es_ref, wih_ref, whh_ref, b_ref, wf_ref, bf_ref, o_ref):
    emb = es_ref[...] * (1.0 / N)
    gx = lax.dot_general(emb, wih_ref[...], (((1,), (1,)), ((), ())),
                         preferred_element_type=jnp.float32)
    b = b_ref[...]
    h = jnp.zeros((1, LH), jnp.float32)
    c = jnp.zeros((1, LH), jnp.float32)
    for t in range(T):
        gh = lax.dot_general(h, whh_ref[...], (((1,), (1,)), ((), ())),
                             preferred_element_type=jnp.float32)
        gates = gx[t:t + 1] + gh + b
        ig = jax.nn.sigmoid(gates[:, 0:LH])
        fg = jax.nn.sigmoid(gates[:, LH:2 * LH])
        gg = jnp.tanh(gates[:, 2 * LH:3 * LH])
        og = jax.nn.sigmoid(gates[:, 3 * LH:4 * LH])
        c = fg * c + ig * gg
        h = og * jnp.tanh(c)
    out = lax.dot_general(h, wf_ref[...], (((1,), (1,)), ((), ())),
                          preferred_element_type=jnp.float32) + bf_ref[...]
    o_ref[...] = out


def _lstm_call(embsum, W_ih, W_hh, bsum, Wf, bfr):
    return pl.pallas_call(
        _lstm_body,
        out_shape=jax.ShapeDtypeStruct((1, A), jnp.float32),
    )(embsum, W_ih, W_hh, bsum, Wf, bfr)


def kernel(x, edge_index, W1, b1, W2, b2, W_ih, W_hh, b_ih, b_hh, Wf, bf):
    src = edge_index[0]
    dst = edge_index[1]
    srcp = jnp.concatenate([src, jnp.zeros((EPAD - E,), jnp.int32)])
    dstp = jnp.concatenate([dst, jnp.full((EPAD - E,), N, jnp.int32)])

    degp = _deg(dstp).reshape(NC, N, 16)
    dinvb = _dinv_call(degp)
    xp = x.reshape(P, 2, N, F_IN)
    g1 = _stage_a_call(xp, W1, dinvb)
    p1 = _prop(g1.reshape(P * N, W), srcp, dstp).reshape(P, N, W)
    b1p = jnp.concatenate([b1, b1]).reshape(1, W)
    g2 = _stage_c_call(p1, g1, dinvb, b1p)
    p2 = _prop(g2.reshape(P * N, W), srcp, dstp).reshape(P, N, W)
    embsum = _stage_e_call(p2, g2, dinvb, W2,
                           b2.reshape(1, H2)).reshape(T, H2)
    return _lstm_call(embsum, W_ih, W_hh,
                      (b_ih + b_hh).reshape(1, 4 * LH), Wf,
                      bf.reshape(1, A))


# speedup vs baseline: 1.7229x; 1.0248x over previous
import functools

import jax
import jax.numpy as jnp
from jax import lax
from jax.experimental import pallas as pl
from jax.experimental.pallas import tpu as pltpu
from jax.experimental.pallas import tpu_sc as plsc

T, N, F_IN = 16, 10000, 256
H1, H2, LH, A = 64, 512, 512, 4
E = 160000

P = T // 2
W = 2 * H1
NC, NS = 2, 16
NW = NC * NS
K = 128
EPAD = 163840
EW = EPAD // NW
NDCHUNK = EW // K
PAIRS0 = 4
ECS = EPAD // NS
NCH = ECS // K
RPS = N // NS
NACC = N + 16
ZR = 25
BM = 1000
NB = N // BM

_mesh = plsc.VectorSubcoreMesh(core_axis_name="c", subcore_axis_name="s")


def _deg_body(dst_hbm, out_hbm, idx_d, ones_v, zbuf, accd, sem):
    c = lax.axis_index("c")
    s = lax.axis_index("s")
    w = s * NC + c
    ebase = w * EW

    def _init(i, _):
        ones_v[i, :] = jnp.ones((16,), jnp.float32)
        zbuf[i, :] = jnp.zeros((16,), jnp.float32)
        return 0
    lax.fori_loop(0, K, _init, 0)

    for k in range(5):
        pltpu.sync_copy(zbuf.at[pl.ds(0, 125)],
                        accd.at[pl.ds(s * RPS + k * 125, 125)])

    @pl.when(s == 0)
    def _():
        pltpu.sync_copy(zbuf.at[pl.ds(0, 16)], accd.at[pl.ds(N, 16)])

    plsc.subcore_barrier()

    def _chunk(i, _):
        pltpu.sync_copy(dst_hbm.at[pl.ds(ebase + i * K, K)], idx_d)
        pltpu.sync_copy(ones_v, accd.at[idx_d], add=True)
        return 0
    lax.fori_loop(0, NDCHUNK, _chunk, 0)

    plsc.subcore_barrier()
    pltpu.sync_copy(accd.at[pl.ds(s * RPS, RPS)], out_hbm.at[c, s])


_deg = functools.partial(
    pl.kernel,
    mesh=_mesh,
    out_type=jax.ShapeDtypeStruct((NC, NS, RPS, 16), jnp.float32),
    scratch_types=[
        pltpu.VMEM((K,), jnp.int32),
        pltpu.VMEM((K, 16), jnp.float32),
        pltpu.VMEM((K, 16), jnp.float32),
        pltpu.VMEM_SHARED((NACC, 16), jnp.float32),
        pltpu.SemaphoreType.DMA,
    ],
)(_deg_body)


def _prop_body(g_hbm, src_hbm, dst_hbm, out_hbm,
               si0, si1, di0, di1, rows_a, rows_b, zbuf, acc,
               isem0, isem1, gsem0, gsem1, osem):
    c = lax.axis_index("c")
    s = lax.axis_index("s")
    ebase = s * ECS
    sbufs = (si0, si1)
    dbufs = (di0, di1)
    isems = (isem0, isem1)
    gsems = (gsem0, gsem1)
    rbufs = (rows_a, rows_b)
    npair = jnp.where(c == 0, PAIRS0, P - PAIRS0)
    pbase = jnp.where(c == 0, 0, PAIRS0)

    def _initz(i, _):
        for j8 in range(W // 16):
            zbuf[i, pl.ds(j8 * 16, 16)] = jnp.zeros((16,), jnp.float32)
        return 0
    lax.fori_loop(0, ZR, _initz, 0)

    def _zero_own_rows():
        for k in range(RPS // ZR):
            pltpu.async_copy(zbuf,
                             acc.at[pl.ds(s * RPS + k * ZR, ZR)], osem)
        for k in range(RPS // ZR):
            pltpu.make_async_copy(zbuf,
                                  acc.at[pl.ds(s * RPS, ZR)], osem).wait()

    _zero_own_rows()

    @pl.when(s == 0)
    def _():
        pltpu.sync_copy(zbuf.at[pl.ds(0, 16)], acc.at[pl.ds(N, 16)])

    plsc.subcore_barrier()

    def _fire_idx(kk, j):
        pltpu.async_copy(src_hbm.at[pl.ds(ebase + kk * K, K)], sbufs[j],
                         isems[j])
        pltpu.async_copy(dst_hbm.at[pl.ds(ebase + kk * K, K)], dbufs[j],
                         isems[j])

    def _wait_idx(j):
        pltpu.make_async_copy(src_hbm.at[pl.ds(ebase, K)], sbufs[j],
                              isems[j]).wait()
        pltpu.make_async_copy(dst_hbm.at[pl.ds(ebase, K)], dbufs[j],
                              isems[j]).wait()

    def _oadd(j, off):
        for m in range(K // 16):
            sl = pl.ds(m * 16, 16)
            sbufs[j][sl] = sbufs[j][sl] + off

    def _prime(p):
        _fire_idx(0, 0)
        _fire_idx(1, 1)
        _wait_idx(0)
        _oadd(0, p * N)
        pltpu.async_copy(g_hbm.at[si0], rows_a, gsem0)

    _prime(pbase)

    def _frame(q, _):
        p = pbase + q
        off = p * N

        def _step(q2, _):
            for j in range(2):
                kk = 2 * q2 + j
                jn = 1 - j

                @pl.when(kk + 1 < NCH)
                def _():
                    _wait_idx(jn)
                    _oadd(jn, off)
                    pltpu.async_copy(g_hbm.at[sbufs[jn]], rbufs[jn],
                                     gsems[jn])

                pltpu.make_async_copy(g_hbm.at[sbufs[j]], rbufs[j],
                                      gsems[j]).wait()
                pltpu.sync_copy(rbufs[j], acc.at[dbufs[j]], add=True)

                @pl.when(kk + 2 < NCH)
                def _():
                    _fire_idx(kk + 2, j)
            return 0
        lax.fori_loop(0, NCH // 2, _step, 0)

        plsc.subcore_barrier()

        @pl.when(q + 1 < npair)
        def _():
            _prime(p + 1)

        pltpu.sync_copy(acc.at[pl.ds(s * RPS, RPS)], out_hbm.at[p, s])
        _zero_own_rows()
        plsc.subcore_barrier()
        return 0
    lax.fori_loop(0, npair, _frame, 0)


_prop = functools.partial(
    pl.kernel,
    mesh=_mesh,
    out_type=jax.ShapeDtypeStruct((P, NS, RPS, W), jnp.float32),
    scratch_types=[
        pltpu.VMEM((K,), jnp.int32),
        pltpu.VMEM((K,), jnp.int32),
        pltpu.VMEM((K,), jnp.int32),
        pltpu.VMEM((K,), jnp.int32),
        pltpu.VMEM((K, W), jnp.float32),
        pltpu.VMEM((K, W), jnp.float32),
        pltpu.VMEM((ZR, W), jnp.float32),
        pltpu.VMEM_SHARED((NACC, W), jnp.float32),
        pltpu.SemaphoreType.DMA,
        pltpu.SemaphoreType.DMA,
        pltpu.SemaphoreType.DMA,
        pltpu.SemaphoreType.DMA,
        pltpu.SemaphoreType.DMA,
    ],
)(_prop_body)


def _dinv_of(degp_ref):
    deg = degp_ref[0, :, 0] + degp_ref[1, :, 0] + 1.0
    return lax.rsqrt(deg)[:, None]


def _stage_a_body(x_ref, w1_ref, degp_ref, o_ref):
    h0 = jnp.dot(x_ref[0, 0], w1_ref[...], preferred_element_type=jnp.float32)
    h1 = jnp.dot(x_ref[0, 1], w1_ref[...], preferred_element_type=jnp.float32)
    o_ref[0] = jnp.concatenate([h0, h1], axis=1) * _dinv_of(degp_ref)


def _stage_a_call(xp, W1, degp):
    return pl.pallas_call(
        _stage_a_body,
        grid=(P, NB),
        in_specs=[
            pl.BlockSpec((1, 2, BM, F_IN), lambda p, j: (p, 0, j, 0)),
            pl.BlockSpec((F_IN, H1), lambda p, j: (0, 0)),
            pl.BlockSpec((NC, BM, 16), lambda p, j: (0, j, 0)),
        ],
        out_specs=pl.BlockSpec((1, BM, W), lambda p, j: (p, j, 0)),
        out_shape=jax.ShapeDtypeStruct((P, N, W), jnp.float32),
    )(xp, W1, degp)


def _stage_c_body(p_ref, g_ref, degp_ref, b1_ref, o_ref):
    dinv = _dinv_of(degp_ref)
    ssum = p_ref[0] + g_ref[0]
    h = jnp.maximum(dinv * ssum + b1_ref[...], 0.0)
    o_ref[0] = dinv * h


def _stage_c_call(p1, g1, degp, b1p):
    return pl.pallas_call(
        _stage_c_body,
        grid=(P, NB),
        in_specs=[
            pl.BlockSpec((1, BM, W), lambda p, j: (p, j, 0)),
            pl.BlockSpec((1, BM, W), lambda p, j: (p, j, 0)),
            pl.BlockSpec((NC, BM, 16), lambda p, j: (0, j, 0)),
            pl.BlockSpec((1, W), lambda p, j: (0, 0)),
        ],
        out_specs=pl.BlockSpec((1, BM, W), lambda p, j: (p, j, 0)),
        out_shape=jax.ShapeDtypeStruct((P, N, W), jnp.float32),
    )(p1, g1, degp, b1p)


def _stage_e_body(p_ref, g_ref, degp_ref, w2_ref, b2_ref,
                  wih_ref, whh_ref, b_ref, wf_ref, bf_ref,
                  es_ref, o_ref):
    pg = pl.program_id(0)
    j = pl.program_id(1)
    s2 = _dinv_of(degp_ref) * (p_ref[0] + g_ref[0])
    ha = jnp.dot(s2[:, :H1], w2_ref[...], preferred_element_type=jnp.float32)
    hb = jnp.dot(s2[:, H1:], w2_ref[...], preferred_element_type=jnp.float32)
    ha = jnp.maximum(ha + b2_ref[...], 0.0)
    hb = jnp.maximum(hb + b2_ref[...], 0.0)
    part = jnp.concatenate([jnp.sum(ha, axis=0, keepdims=True),
                            jnp.sum(hb, axis=0, keepdims=True)],
                           axis=0)[None]

    @pl.when(j == 0)
    def _():
        es_ref[pl.ds(pg, 1)] = part

    @pl.when(j > 0)
    def _():
        es_ref[pl.ds(pg, 1)] += part

    @pl.when((pg == P - 1) & (j == NB - 1))
    def _():
        emb = es_ref[...].reshape(T, H2) * (1.0 / N)
        gx = lax.dot_general(emb, wih_ref[...], (((1,), (1,)), ((), ())),
                             preferred_element_type=jnp.float32)
        b = b_ref[...]
        h = jnp.zeros((1, LH), jnp.float32)
        c = jnp.zeros((1, LH), jnp.float32)
        for t in range(T):
            gh = lax.dot_general(h, whh_ref[...], (((1,), (1,)), ((), ())),
                                 preferred_element_type=jnp.float32)
            gates = gx[t:t + 1] + gh + b
            ig = jax.nn.sigmoid(gates[:, 0:LH])
            fg = jax.nn.sigmoid(gates[:, LH:2 * LH])
            gg = jnp.tanh(gates[:, 2 * LH:3 * LH])
            og = jax.nn.sigmoid(gates[:, 3 * LH:4 * LH])
            c = fg * c + ig * gg
            h = og * jnp.tanh(c)
        out = lax.dot_general(h, wf_ref[...], (((1,), (1,)), ((), ())),
                              preferred_element_type=jnp.float32)
        o_ref[...] = out + bf_ref[...]


def _stage_e_call(p2, g2, degp, W2, b2r, W_ih, W_hh, bsum, Wf, bfr):
    _, out = pl.pallas_call(
        _stage_e_body,
        grid=(P, NB),
        in_specs=[
            pl.BlockSpec((1, BM, W), lambda p, j: (p, j, 0)),
            pl.BlockSpec((1, BM, W), lambda p, j: (p, j, 0)),
            pl.BlockSpec((NC, BM, 16), lambda p, j: (0, j, 0)),
            pl.BlockSpec((H1, H2), lambda p, j: (0, 0)),
            pl.BlockSpec((1, H2), lambda p, j: (0, 0)),
            pl.BlockSpec((4 * LH, H2), lambda p, j: (0, 0)),
            pl.BlockSpec((4 * LH, LH), lambda p, j: (0, 0)),
            pl.BlockSpec((1, 4 * LH), lambda p, j: (0, 0)),
            pl.BlockSpec((A, LH), lambda p, j: (0, 0)),
            pl.BlockSpec((1, A), lambda p, j: (0, 0)),
        ],
        out_specs=[
            pl.BlockSpec((P, 2, H2), lambda p, j: (0, 0, 0)),
            pl.BlockSpec((1, A), lambda p, j: (0, 0)),
        ],
        out_shape=[
            jax.ShapeDtypeStruct((P, 2, H2), jnp.float32),
            jax.ShapeDtypeStruct((1, A), jnp.float32),
        ],
    )(p2, g2, degp, W2, b2r, W_ih, W_hh, bsum, Wf, bfr)
    return out


def kernel(x, edge_index, W1, b1, W2, b2, W_ih, W_hh, b_ih, b_hh, Wf, bf):
    src = edge_index[0]
    dst = edge_index[1]
    srcp = jnp.concatenate([src, jnp.zeros((EPAD - E,), jnp.int32)])
    dstp = jnp.concatenate([dst, jnp.full((EPAD - E,), N, jnp.int32)])

    degp = _deg(dstp).reshape(NC, N, 16)
    xp = x.reshape(P, 2, N, F_IN)
    g1 = _stage_a_call(xp, W1, degp)
    p1 = _prop(g1.reshape(P * N, W), srcp, dstp).reshape(P, N, W)
    b1p = jnp.concatenate([b1, b1]).reshape(1, W)
    g2 = _stage_c_call(p1, g1, degp, b1p)
    p2 = _prop(g2.reshape(P * N, W), srcp, dstp).reshape(P, N, W)
    return _stage_e_call(p2, g2, degp, W2, b2.reshape(1, H2),
                         W_ih, W_hh, (b_ih + b_hh).reshape(1, 4 * LH),
                         Wf, bf.reshape(1, A))
